# Initial kernel scaffold; baseline (speedup 1.0000x reference)
import jax
import jax.numpy as jnp
from jax.experimental import pallas as pl


def kernel(x, edge_index, edge_attr, W_root, b_root, W1, b1, W2, b2):
    raise NotImplementedError("write your pallas kernel here")


# speedup vs baseline: 3.0444x; 3.0444x over previous
import functools

import jax
import jax.numpy as jnp
from jax import lax
from jax.experimental import pallas as pl
from jax.experimental.pallas import tpu as pltpu
from jax.experimental.pallas import tpu_sc as plsc

N_NODES = 10000
E_EDGES = 160000
IN = 32
OUT = 32
ED = 16
EH = 32

NC = 2
NS = 16
NW = NC * NS

CHUNK = 128
NCHUNK = E_EDGES // CHUNK
ITERS = (NCHUNK + NW - 1) // NW

ROW = 48
NPAD = 10240
SLICE = NPAD // NS

TE = 640
TN = 2000

def _sc_mesh():
    return plsc.VectorSubcoreMesh(
        core_axis_name="c", subcore_axis_name="s",
        num_cores=NC, num_subcores=NS)


@functools.lru_cache(maxsize=None)
def _build_gather():
    @functools.partial(
        pl.kernel,
        out_type=jax.ShapeDtypeStruct((E_EDGES, IN), jnp.float32),
        mesh=_sc_mesh(),
        compiler_params=pltpu.CompilerParams(use_tc_tiling_on_sc=False),
        scratch_types=[
            pltpu.VMEM((CHUNK,), jnp.int32),
            pltpu.VMEM((CHUNK, IN), jnp.float32),
            pltpu.SemaphoreType.DMA,
        ],
    )
    def gather(x_hbm, src_hbm, out_hbm, idx_v, rows_v, sem):
        cid = lax.axis_index("c")
        sid = lax.axis_index("s")
        wid = sid * NC + cid

        def body(j, carry):
            c = wid + j * NW

            @pl.when(c < NCHUNK)
            def _():
                base = c * CHUNK
                pltpu.sync_copy(src_hbm.at[pl.ds(base, CHUNK)], idx_v)
                pltpu.async_copy(x_hbm.at[idx_v], rows_v, sem).wait()
                pltpu.sync_copy(rows_v, out_hbm.at[pl.ds(base, CHUNK)])

            return carry

        lax.fori_loop(0, ITERS, body, 0)

    return gather


@functools.lru_cache(maxsize=None)
def _build_scatter():
    @functools.partial(
        pl.kernel,
        out_type=jax.ShapeDtypeStruct((NC * NPAD, ROW), jnp.float32),
        mesh=_sc_mesh(),
        compiler_params=pltpu.CompilerParams(use_tc_tiling_on_sc=False),
        scratch_types=[
            pltpu.VMEM((CHUNK,), jnp.int32),
            pltpu.VMEM((CHUNK, ROW), jnp.float32),
            pltpu.VMEM_SHARED((NPAD, ROW), jnp.float32),
        ],
    )
    def scatter(msg_hbm, dst_hbm, zeros_hbm, out_hbm, dst_v, val_v, acc_sh):
        cid = lax.axis_index("c")
        sid = lax.axis_index("s")
        wid = sid * NC + cid

        pltpu.sync_copy(zeros_hbm, acc_sh.at[pl.ds(sid * SLICE, SLICE)])
        plsc.subcore_barrier()

        def body(j, carry):
            c = wid + j * NW

            @pl.when(c < NCHUNK)
            def _():
                base = c * CHUNK
                pltpu.sync_copy(dst_hbm.at[pl.ds(base, CHUNK)], dst_v)
                pltpu.sync_copy(msg_hbm.at[pl.ds(base, CHUNK)], val_v)
                pltpu.sync_copy(val_v, acc_sh.at[dst_v], add=True)

            return carry

        lax.fori_loop(0, ITERS, body, 0)
        plsc.subcore_barrier()
        pltpu.sync_copy(acc_sh.at[pl.ds(sid * SLICE, SLICE)],
                        out_hbm.at[pl.ds(cid * NPAD + sid * SLICE, SLICE)])

    return scatter


def _msg_body(ea_ref, xj_ref, w1_ref, b1_ref, rh_ref, m_ref, b2r_ref, out_ref):
    ea = ea_ref[...]
    xj = xj_ref[...]
    h = jnp.maximum(
        jnp.dot(ea, w1_ref[...], preferred_element_type=jnp.float32)
        + b1_ref[...], 0.0)
    hexp = jnp.dot(h, rh_ref[...], preferred_element_type=jnp.float32)
    xrep = pltpu.repeat(xj, EH, axis=1)
    msg = (jnp.dot(hexp * xrep, m_ref[...], preferred_element_type=jnp.float32)
           + jnp.dot(xj, b2r_ref[...], preferred_element_type=jnp.float32))
    col = lax.broadcasted_iota(jnp.int32, (TE, ROW - OUT), 1)
    count_lane = (col == 0).astype(jnp.float32)
    out_ref[...] = jnp.concatenate([msg, count_lane], axis=1)


def _tc_msg(edge_attr, xj, W1, b1r, Rh, M, B2r):
    grid = (E_EDGES // TE,)
    return pl.pallas_call(
        _msg_body,
        grid=grid,
        in_specs=[
            pl.BlockSpec((TE, ED), lambda e: (e, 0)),
            pl.BlockSpec((TE, IN), lambda e: (e, 0)),
            pl.BlockSpec((ED, EH), lambda e: (0, 0)),
            pl.BlockSpec((1, EH), lambda e: (0, 0)),
            pl.BlockSpec((EH, EH * IN), lambda e: (0, 0)),
            pl.BlockSpec((EH * IN, OUT), lambda e: (0, 0)),
            pl.BlockSpec((IN, OUT), lambda e: (0, 0)),
        ],
        out_specs=pl.BlockSpec((TE, ROW), lambda e: (e, 0)),
        out_shape=jax.ShapeDtypeStruct((E_EDGES, ROW), jnp.float32),
    )(edge_attr, xj, W1, b1r, Rh, M, B2r)


def _finish_body(p_ref, x_ref, wr_ref, br_ref, out_ref):
    p = p_ref[...]
    s = p[0] + p[1]
    sums = s[:, :OUT]
    cnt = s[:, OUT:OUT + 1]
    mean = sums / jnp.maximum(cnt, 1.0)
    root = jnp.dot(x_ref[...], wr_ref[...], preferred_element_type=jnp.float32)
    out_ref[...] = jnp.maximum(root + br_ref[...] + mean, 0.0)


def _tc_finish(part, x, W_root, b_rootr):
    grid = (N_NODES // TN,)
    return pl.pallas_call(
        _finish_body,
        grid=grid,
        in_specs=[
            pl.BlockSpec((NC, TN, ROW), lambda i: (0, i, 0)),
            pl.BlockSpec((TN, IN), lambda i: (i, 0)),
            pl.BlockSpec((IN, OUT), lambda i: (0, 0)),
            pl.BlockSpec((1, OUT), lambda i: (0, 0)),
        ],
        out_specs=pl.BlockSpec((TN, OUT), lambda i: (i, 0)),
        out_shape=jax.ShapeDtypeStruct((N_NODES, OUT), jnp.float32),
    )(part, x, W_root, b_rootr)


def kernel(x, edge_index, edge_attr, W_root, b_root, W1, b1, W2, b2):
    src = edge_index[0]
    dst = edge_index[1]

    xj = _build_gather()(x, src)

    Rh = jnp.repeat(jnp.eye(EH, dtype=jnp.float32), IN, axis=1)
    M = W2.reshape(EH, IN, OUT).reshape(EH * IN, OUT)
    B2r = b2.reshape(IN, OUT)
    msg48 = _tc_msg(edge_attr, xj, W1, b1.reshape(1, EH), Rh, M, B2r)

    zeros = jnp.zeros((SLICE, ROW), dtype=jnp.float32)
    part = _build_scatter()(msg48, dst, zeros)

    out = _tc_finish(part.reshape(NC, NPAD, ROW), x, W_root,
                     b_root.reshape(1, OUT))
    return out


# speedup vs baseline: 4.7817x; 1.5707x over previous
import functools

import jax
import jax.numpy as jnp
from jax import lax
from jax.experimental import pallas as pl
from jax.experimental.pallas import tpu as pltpu
from jax.experimental.pallas import tpu_sc as plsc

N_NODES = 10000
E_EDGES = 160000
IN = 32
OUT = 32
ED = 16
EH = 32

NC = 2
NS = 16
NW = NC * NS

CHUNK = 128
NCHUNK = E_EDGES // CHUNK
ITERS = (NCHUNK + NW - 1) // NW

LANES = 128
ROW = 64
NPAD = 10240
SLICE = NPAD // NS

TE = 1600
TN = 2000


def _sc_mesh():
    return plsc.VectorSubcoreMesh(
        core_axis_name="c", subcore_axis_name="s",
        num_cores=NC, num_subcores=NS)


@functools.lru_cache(maxsize=None)
def _build_gather():
    @functools.partial(
        pl.kernel,
        out_type=jax.ShapeDtypeStruct((E_EDGES, LANES), jnp.float32),
        mesh=_sc_mesh(),
        compiler_params=pltpu.CompilerParams(use_tc_tiling_on_sc=False),
        scratch_types=[
            pltpu.VMEM((CHUNK,), jnp.int32),
            pltpu.VMEM((CHUNK, IN), jnp.float32),
            pltpu.SemaphoreType.DMA,
        ],
    )
    def gather(x_hbm, src_hbm, out_hbm, idx_v, rows_v, sem):
        cid = lax.axis_index("c")
        sid = lax.axis_index("s")
        wid = sid * NC + cid

        def body(j, carry):
            c = wid + j * NW

            @pl.when(c < NCHUNK)
            def _():
                base = c * CHUNK
                pltpu.sync_copy(src_hbm.at[pl.ds(base, CHUNK)], idx_v)
                pltpu.async_copy(x_hbm.at[idx_v], rows_v, sem).wait()
                pltpu.sync_copy(
                    rows_v, out_hbm.at[pl.ds(base, CHUNK), pl.ds(0, IN)])

            return carry

        lax.fori_loop(0, ITERS, body, 0)

    return gather


@functools.lru_cache(maxsize=None)
def _build_scatter():
    @functools.partial(
        pl.kernel,
        out_type=jax.ShapeDtypeStruct((NC * NPAD, LANES), jnp.float32),
        mesh=_sc_mesh(),
        compiler_params=pltpu.CompilerParams(use_tc_tiling_on_sc=False),
        scratch_types=[
            pltpu.VMEM((CHUNK,), jnp.int32),
            pltpu.VMEM((CHUNK, ROW), jnp.float32),
            pltpu.VMEM_SHARED((NPAD, ROW), jnp.float32),
        ],
    )
    def scatter(msg_hbm, dst_hbm, zeros_hbm, out_hbm, dst_v, val_v, acc_sh):
        cid = lax.axis_index("c")
        sid = lax.axis_index("s")
        wid = sid * NC + cid

        pltpu.sync_copy(zeros_hbm, acc_sh.at[pl.ds(sid * SLICE, SLICE)])
        plsc.subcore_barrier()

        def body(j, carry):
            c = wid + j * NW

            @pl.when(c < NCHUNK)
            def _():
                base = c * CHUNK
                pltpu.sync_copy(dst_hbm.at[pl.ds(base, CHUNK)], dst_v)
                pltpu.sync_copy(
                    msg_hbm.at[pl.ds(base, CHUNK), pl.ds(0, ROW)], val_v)
                pltpu.sync_copy(val_v, acc_sh.at[dst_v], add=True)

            return carry

        lax.fori_loop(0, ITERS, body, 0)
        plsc.subcore_barrier()
        pltpu.sync_copy(
            acc_sh.at[pl.ds(sid * SLICE, SLICE)],
            out_hbm.at[pl.ds(cid * NPAD + sid * SLICE, SLICE), pl.ds(0, ROW)])

    return scatter


def _msg_body(ea_ref, xj_ref, w1e_ref, b1e_ref, m_ref, b2r_ref, out_ref):
    ea = ea_ref[...]
    xj = xj_ref[:, :IN]
    hexp = jnp.maximum(
        jnp.dot(ea, w1e_ref[...],
                preferred_element_type=jnp.float32).astype(jnp.bfloat16)
        + b1e_ref[...], jnp.bfloat16(0.0))
    xrep = pltpu.repeat(xj.astype(jnp.bfloat16), EH, axis=1)
    z = hexp * xrep
    msg = (jnp.dot(z, m_ref[...], preferred_element_type=jnp.float32)
           + jnp.dot(xj, b2r_ref[...], preferred_element_type=jnp.float32))
    col = lax.broadcasted_iota(jnp.int32, (TE, ROW - OUT), 1)
    count_lane = (col == 0).astype(jnp.float32)
    out_ref[:, :ROW] = jnp.concatenate([msg, count_lane], axis=1)


def _tc_msg(edge_attr, xj128, W1e, b1e, M, B2r):
    grid = (E_EDGES // TE,)
    return pl.pallas_call(
        _msg_body,
        grid=grid,
        in_specs=[
            pl.BlockSpec((TE, ED), lambda e: (e, 0)),
            pl.BlockSpec((TE, LANES), lambda e: (e, 0)),
            pl.BlockSpec((ED, EH * IN), lambda e: (0, 0)),
            pl.BlockSpec((1, EH * IN), lambda e: (0, 0)),
            pl.BlockSpec((EH * IN, OUT), lambda e: (0, 0)),
            pl.BlockSpec((IN, OUT), lambda e: (0, 0)),
        ],
        out_specs=pl.BlockSpec((TE, LANES), lambda e: (e, 0)),
        out_shape=jax.ShapeDtypeStruct((E_EDGES, LANES), jnp.float32),
    )(edge_attr, xj128, W1e, b1e, M, B2r)


def _finish_body(p_ref, x_ref, wr_ref, br_ref, out_ref):
    p = p_ref[:, :, :ROW]
    s = p[0] + p[1]
    sums = s[:, :OUT]
    cnt = s[:, OUT:OUT + 1]
    mean = sums / jnp.maximum(cnt, 1.0)
    root = jnp.dot(x_ref[...], wr_ref[...], preferred_element_type=jnp.float32)
    out_ref[...] = jnp.maximum(root + br_ref[...] + mean, 0.0)


def _tc_finish(part, x, W_root, b_rootr):
    grid = (N_NODES // TN,)
    return pl.pallas_call(
        _finish_body,
        grid=grid,
        in_specs=[
            pl.BlockSpec((NC, TN, LANES), lambda i: (0, i, 0)),
            pl.BlockSpec((TN, IN), lambda i: (i, 0)),
            pl.BlockSpec((IN, OUT), lambda i: (0, 0)),
            pl.BlockSpec((1, OUT), lambda i: (0, 0)),
        ],
        out_specs=pl.BlockSpec((TN, OUT), lambda i: (i, 0)),
        out_shape=jax.ShapeDtypeStruct((N_NODES, OUT), jnp.float32),
    )(part, x, W_root, b_rootr)


def kernel(x, edge_index, edge_attr, W_root, b_root, W1, b1, W2, b2):
    src = edge_index[0]
    dst = edge_index[1]

    xj128 = _build_gather()(x, src)

    Rh = jnp.repeat(jnp.eye(EH, dtype=jnp.float32), IN, axis=1)
    W1e = (W1 @ Rh).astype(jnp.bfloat16)
    b1e = (b1 @ Rh).reshape(1, EH * IN).astype(jnp.bfloat16)
    ea16 = edge_attr.astype(jnp.bfloat16)
    M = W2.reshape(EH, IN, OUT).reshape(EH * IN, OUT).astype(jnp.bfloat16)
    B2r = b2.reshape(IN, OUT)
    msg128 = _tc_msg(ea16, xj128, W1e, b1e, M, B2r)

    zeros = jnp.zeros((SLICE, ROW), dtype=jnp.float32)
    part = _build_scatter()(msg128, dst, zeros)

    out = _tc_finish(part.reshape(NC, NPAD, LANES), x, W_root,
                     b_root.reshape(1, OUT))
    return out


# speedup vs baseline: 5.0627x; 1.0588x over previous
import functools

import jax
import jax.numpy as jnp
from jax import lax
from jax.experimental import pallas as pl
from jax.experimental.pallas import tpu as pltpu
from jax.experimental.pallas import tpu_sc as plsc

N_NODES = 10000
E_EDGES = 160000
IN = 32
OUT = 32
ED = 16
EH = 32

NC = 2
NS = 16
NW = NC * NS

NSEG = 2
ESEG = E_EDGES // NSEG

CHUNK = 128

LANES = 128
ROW = 64
NPAD = 10240
SLICE = NPAD // NS

TE = 1600
TN = 2000


def _sc_mesh():
    return plsc.VectorSubcoreMesh(
        core_axis_name="c", subcore_axis_name="s",
        num_cores=NC, num_subcores=NS)


@functools.lru_cache(maxsize=None)
def _build_gather(n_edges):
    nchunk = n_edges // CHUNK
    iters = (nchunk + NW - 1) // NW

    @functools.partial(
        pl.kernel,
        out_type=jax.ShapeDtypeStruct((n_edges, LANES), jnp.float32),
        mesh=_sc_mesh(),
        compiler_params=pltpu.CompilerParams(use_tc_tiling_on_sc=False),
        scratch_types=[
            pltpu.VMEM((CHUNK,), jnp.int32),
            pltpu.VMEM((CHUNK, IN), jnp.float32),
            pltpu.SemaphoreType.DMA,
        ],
    )
    def gather(x_hbm, src_hbm, out_hbm, idx_v, rows_v, sem):
        cid = lax.axis_index("c")
        sid = lax.axis_index("s")
        wid = sid * NC + cid

        def body(j, carry):
            c = wid + j * NW

            @pl.when(c < nchunk)
            def _():
                base = c * CHUNK
                pltpu.sync_copy(src_hbm.at[pl.ds(base, CHUNK)], idx_v)
                pltpu.async_copy(x_hbm.at[idx_v], rows_v, sem).wait()
                pltpu.sync_copy(
                    rows_v, out_hbm.at[pl.ds(base, CHUNK), pl.ds(0, IN)])

            return carry

        lax.fori_loop(0, iters, body, 0)

    return gather


@functools.lru_cache(maxsize=None)
def _build_scatter(n_edges):
    nchunk = n_edges // CHUNK
    iters = (nchunk + NW - 1) // NW

    @functools.partial(
        pl.kernel,
        out_type=jax.ShapeDtypeStruct((NC * NPAD, LANES), jnp.float32),
        mesh=_sc_mesh(),
        compiler_params=pltpu.CompilerParams(use_tc_tiling_on_sc=False),
        scratch_types=[
            pltpu.VMEM((CHUNK,), jnp.int32),
            pltpu.VMEM((CHUNK, ROW), jnp.float32),
            pltpu.VMEM_SHARED((NPAD, ROW), jnp.float32),
        ],
    )
    def scatter(msg_hbm, dst_hbm, zeros_hbm, out_hbm, dst_v, val_v, acc_sh):
        cid = lax.axis_index("c")
        sid = lax.axis_index("s")
        wid = sid * NC + cid

        pltpu.sync_copy(zeros_hbm, acc_sh.at[pl.ds(sid * SLICE, SLICE)])
        plsc.subcore_barrier()

        def body(j, carry):
            c = wid + j * NW

            @pl.when(c < nchunk)
            def _():
                base = c * CHUNK
                pltpu.sync_copy(dst_hbm.at[pl.ds(base, CHUNK)], dst_v)
                pltpu.sync_copy(
                    msg_hbm.at[pl.ds(base, CHUNK), pl.ds(0, ROW)], val_v)
                pltpu.sync_copy(val_v, acc_sh.at[dst_v], add=True)

            return carry

        lax.fori_loop(0, iters, body, 0)
        plsc.subcore_barrier()
        pltpu.sync_copy(
            acc_sh.at[pl.ds(sid * SLICE, SLICE)],
            out_hbm.at[pl.ds(cid * NPAD + sid * SLICE, SLICE), pl.ds(0, ROW)])

    return scatter


def _msg_body(ea_ref, xj_ref, w1e_ref, b1e_ref, m_ref, b2r_ref, out_ref):
    ea = ea_ref[...]
    xj = xj_ref[:, :IN].astype(jnp.bfloat16)
    hexp = jnp.maximum(
        jnp.dot(ea, w1e_ref[...],
                preferred_element_type=jnp.float32).astype(jnp.bfloat16)
        + b1e_ref[...], jnp.bfloat16(0.0))
    xrep = pltpu.repeat(xj, EH, axis=1)
    z = hexp * xrep
    msg = (jnp.dot(z, m_ref[...], preferred_element_type=jnp.float32)
           + jnp.dot(xj, b2r_ref[...], preferred_element_type=jnp.float32))
    col = lax.broadcasted_iota(jnp.int32, (TE, ROW - OUT), 1)
    count_lane = (col == 0).astype(jnp.float32)
    out_ref[:, :ROW] = jnp.concatenate([msg, count_lane], axis=1)


def _tc_msg(edge_attr, xj128, W1e, b1e, M, B2r):
    n_edges = edge_attr.shape[0]
    grid = (n_edges // TE,)
    return pl.pallas_call(
        _msg_body,
        grid=grid,
        in_specs=[
            pl.BlockSpec((TE, ED), lambda e: (e, 0)),
            pl.BlockSpec((TE, LANES), lambda e: (e, 0)),
            pl.BlockSpec((ED, EH * IN), lambda e: (0, 0)),
            pl.BlockSpec((1, EH * IN), lambda e: (0, 0)),
            pl.BlockSpec((EH * IN, OUT), lambda e: (0, 0)),
            pl.BlockSpec((IN, OUT), lambda e: (0, 0)),
        ],
        out_specs=pl.BlockSpec((TE, LANES), lambda e: (e, 0)),
        out_shape=jax.ShapeDtypeStruct((n_edges, LANES), jnp.float32),
    )(edge_attr, xj128, W1e, b1e, M, B2r)


def _finish_body(p0_ref, p1_ref, x_ref, wr_ref, br_ref, out_ref):
    s = (p0_ref[0, :, :ROW] + p0_ref[1, :, :ROW]
         + p1_ref[0, :, :ROW] + p1_ref[1, :, :ROW])
    sums = s[:, :OUT]
    cnt = s[:, OUT:OUT + 1]
    mean = sums / jnp.maximum(cnt, 1.0)
    root = jnp.dot(x_ref[...], wr_ref[...], preferred_element_type=jnp.float32)
    out_ref[...] = jnp.maximum(root + br_ref[...] + mean, 0.0)


def _tc_finish(parts, x, W_root, b_rootr):
    grid = (N_NODES // TN,)
    pspec = pl.BlockSpec((NC, TN, LANES), lambda i: (0, i, 0))
    return pl.pallas_call(
        _finish_body,
        grid=grid,
        in_specs=[
            pspec,
            pspec,
            pl.BlockSpec((TN, IN), lambda i: (i, 0)),
            pl.BlockSpec((IN, OUT), lambda i: (0, 0)),
            pl.BlockSpec((1, OUT), lambda i: (0, 0)),
        ],
        out_specs=pl.BlockSpec((TN, OUT), lambda i: (i, 0)),
        out_shape=jax.ShapeDtypeStruct((N_NODES, OUT), jnp.float32),
    )(parts[0], parts[1], x, W_root, b_rootr)


def kernel(x, edge_index, edge_attr, W_root, b_root, W1, b1, W2, b2):
    src = edge_index[0]
    dst = edge_index[1]

    Rh = jnp.repeat(jnp.eye(EH, dtype=jnp.float32), IN, axis=1)
    W1e = (W1 @ Rh).astype(jnp.bfloat16)
    b1e = (b1 @ Rh).reshape(1, EH * IN).astype(jnp.bfloat16)
    ea16 = edge_attr.astype(jnp.bfloat16)
    M = W2.reshape(EH, IN, OUT).reshape(EH * IN, OUT).astype(jnp.bfloat16)
    B2r = b2.reshape(IN, OUT).astype(jnp.bfloat16)
    zeros = jnp.zeros((SLICE, ROW), dtype=jnp.float32)

    gather = _build_gather(ESEG)
    scatter = _build_scatter(ESEG)

    parts = []
    for s in range(NSEG):
        lo = s * ESEG
        xj128 = gather(x, lax.slice(src, (lo,), (lo + ESEG,)))
        msg128 = _tc_msg(lax.slice(ea16, (lo, 0), (lo + ESEG, ED)),
                         xj128, W1e, b1e, M, B2r)
        parts.append(scatter(msg128,
                             lax.slice(dst, (lo,), (lo + ESEG,)), zeros))

    out = _tc_finish([p.reshape(NC, NPAD, LANES) for p in parts],
                     x, W_root, b_root.reshape(1, OUT))
    return out
